# Initial kernel scaffold; baseline (speedup 1.0000x reference)
import jax
import jax.numpy as jnp
from jax.experimental import pallas as pl


def kernel(x, edge_index, Ws1, Wn1, b1, Ws2, Wn2, b2):
    raise NotImplementedError("write your pallas kernel here")


# speedup vs baseline: 3.3984x; 3.3984x over previous
import functools
import math

import jax
import jax.numpy as jnp
from jax import lax
from jax.experimental import pallas as pl
from jax.experimental.pallas import tpu as pltpu
from jax.experimental.pallas import tpu_sc as plsc

B = 128
KI = 8
NC = 2
NS = 16
NW = NC * NS
ZR = 8


def _sc_agg_body(nb, npad, d, x_hbm, src_hbm, dst_hbm, zrow_hbm,
                 agg_out, agg_sh, idx_src, idx_dst, rows, zbuf, sem):
    cid = lax.axis_index("c")
    sid = lax.axis_index("s")
    wid = cid * NS + sid
    stripe = npad // NS
    lo = pl.multiple_of(sid * stripe, ZR)

    pltpu.sync_copy(zrow_hbm, zbuf)

    def zbody(k, c):
        off = pl.multiple_of(lo + k * ZR, ZR)
        pltpu.sync_copy(zbuf, agg_sh.at[pl.ds(off, ZR)])
        return c

    lax.fori_loop(0, stripe // ZR, zbody, 0)
    plsc.subcore_barrier()

    def chunk_body(c, carry):
        coff = pl.multiple_of(c * KI, KI)
        pltpu.sync_copy(src_hbm.at[wid, pl.ds(coff, KI)], idx_src)
        pltpu.sync_copy(dst_hbm.at[wid, pl.ds(coff, KI)], idx_dst)
        def body(j, c2):
            pltpu.async_copy(x_hbm.at[idx_src.at[j]], rows, sem).wait()
            pltpu.sync_copy(rows, agg_sh.at[idx_dst.at[j]], add=True)
            return c2
        lax.fori_loop(0, KI, body, 0)
        return carry

    lax.fori_loop(0, nb // KI, chunk_body, 0)
    plsc.subcore_barrier()

    pltpu.sync_copy(agg_sh.at[pl.ds(lo, stripe)],
                    agg_out.at[cid, pl.ds(lo, stripe)])


def _make_sc_agg(nb, npad, d):
    mesh = plsc.VectorSubcoreMesh(core_axis_name="c", subcore_axis_name="s")
    return pl.kernel(
        functools.partial(_sc_agg_body, nb, npad, d),
        out_type=jax.ShapeDtypeStruct((NC, npad, d), jnp.float32),
        mesh=mesh,
        scratch_types=[
            pltpu.VMEM_SHARED((npad, d), jnp.float32),
            pltpu.VMEM((KI, B), jnp.int32),
            pltpu.VMEM((KI, B), jnp.int32),
            pltpu.VMEM((B, d), jnp.float32),
            pltpu.VMEM((ZR, d), jnp.float32),
            pltpu.SemaphoreType.DMA,
        ],
    )


def _sc_cnt_body(nb, npad, d, dst_hbm, zrow_hbm, ones_hbm,
                 cnt_out, cnt_sh, idx_dst, ones_v, zcbuf):
    cid = lax.axis_index("c")
    sid = lax.axis_index("s")
    wid = cid * NS + sid
    stripe = npad // NS
    lo = pl.multiple_of(sid * stripe, ZR)

    pltpu.sync_copy(zrow_hbm, zcbuf)
    pltpu.sync_copy(ones_hbm, ones_v)
    pltpu.sync_copy(dst_hbm.at[wid], idx_dst)

    def zbody(k, c):
        off = pl.multiple_of(lo + k * ZR, ZR)
        pltpu.sync_copy(zcbuf, cnt_sh.at[pl.ds(off, ZR)])
        return c

    lax.fori_loop(0, stripe // ZR, zbody, 0)
    plsc.subcore_barrier()

    def body(j, c2):
        pltpu.sync_copy(ones_v, cnt_sh.at[idx_dst.at[j]], add=True)
        return c2

    lax.fori_loop(0, nb, body, 0)
    plsc.subcore_barrier()
    pltpu.sync_copy(cnt_sh.at[pl.ds(lo, stripe)],
                    cnt_out.at[cid, pl.ds(lo, stripe)])


def _make_sc_cnt(nb, npad, d):
    mesh = plsc.VectorSubcoreMesh(core_axis_name="c", subcore_axis_name="s")
    return pl.kernel(
        functools.partial(_sc_cnt_body, nb, npad, d),
        out_type=jax.ShapeDtypeStruct((NC, npad, d), jnp.float32),
        mesh=mesh,
        scratch_types=[
            pltpu.VMEM_SHARED((npad, d), jnp.float32),
            pltpu.VMEM((nb, B), jnp.int32),
            pltpu.VMEM((B, d), jnp.float32),
            pltpu.VMEM((ZR, d), jnp.float32),
        ],
    )


def _tc_layer_body(relu, x_ref, aggp_ref, cntp_ref, ws_ref, wn_ref, b_ref,
                   out_ref):
    agg = aggp_ref[0] + aggp_ref[1]
    cnt = cntp_ref[0][:, 0:1] + cntp_ref[1][:, 0:1]
    mean = agg / jnp.maximum(cnt, 1.0)
    h = (jnp.dot(x_ref[...], ws_ref[...], preferred_element_type=jnp.float32)
         + jnp.dot(mean, wn_ref[...], preferred_element_type=jnp.float32)
         + b_ref[...])
    if relu:
        h = jnp.maximum(h, 0.0)
    out_ref[...] = h


def _tc_layer(x, agg_p, cnt_p, ws, wn, b, relu):
    n, d = x.shape
    blk = 2000 if n % 2000 == 0 else n
    grid = (n // blk,)
    return pl.pallas_call(
        functools.partial(_tc_layer_body, relu),
        grid=grid,
        in_specs=[
            pl.BlockSpec((blk, d), lambda i: (i, 0)),
            pl.BlockSpec((NC, blk, d), lambda i: (0, i, 0)),
            pl.BlockSpec((NC, blk, d), lambda i: (0, i, 0)),
            pl.BlockSpec((d, d), lambda i: (0, 0)),
            pl.BlockSpec((d, d), lambda i: (0, 0)),
            pl.BlockSpec((1, d), lambda i: (0, 0)),
        ],
        out_specs=pl.BlockSpec((blk, d), lambda i: (i, 0)),
        out_shape=jax.ShapeDtypeStruct((n, d), jnp.float32),
    )(x, agg_p, cnt_p, ws, wn, b.reshape(1, d))


def kernel(x, edge_index, Ws1, Wn1, b1, Ws2, Wn2, b2):
    n, d = x.shape
    e = edge_index.shape[1]
    per_tile = math.ceil(e / NW)
    nb = math.ceil(per_tile / B)
    nb = ((nb + KI - 1) // KI) * KI
    cap = NW * nb * B
    npad = ((n + 8 + NS * 8 - 1) // (NS * 8)) * (NS * 8)
    stripe = npad // NS

    src = edge_index[0]
    dst = edge_index[1]
    pad = cap - e
    src_r = jnp.pad(src, (0, pad)).reshape(NW, nb, B)
    dst_r = jnp.pad(dst, (0, pad), constant_values=npad - 1).reshape(NW, nb, B)

    zrow = jnp.zeros((ZR, d), jnp.float32)
    ones = jnp.ones((B, d), jnp.float32)

    sc_agg = _make_sc_agg(nb, npad, d)
    sc_cnt = _make_sc_cnt(nb, npad, d)

    cnt_p = sc_cnt(dst_r, zrow, ones)
    agg1_p = sc_agg(x, src_r, dst_r, zrow)
    h = _tc_layer(x, agg1_p, cnt_p, Ws1, Wn1, b1, relu=True)
    agg2_p = sc_agg(h, src_r, dst_r, zrow)
    out = _tc_layer(h, agg2_p, cnt_p, Ws2, Wn2, b2, relu=False)
    return out


# speedup vs baseline: 3.7331x; 1.0985x over previous
import functools
import math

import jax
import jax.numpy as jnp
from jax import lax
from jax.experimental import pallas as pl
from jax.experimental.pallas import tpu as pltpu
from jax.experimental.pallas import tpu_sc as plsc

B = 128
KI = 8
NC = 2
NS = 16
NW = NC * NS
ZR = 8


def _sc_agg_body(nb, npad, d, x_hbm, src_hbm, dst_hbm, zrow_hbm,
                 agg_out, agg_sh, idx_src, idx_dst, rows0, rows1, zbuf,
                 sem0, sem1):
    cid = lax.axis_index("c")
    sid = lax.axis_index("s")
    wid = cid * NS + sid
    stripe = npad // NS
    lo = pl.multiple_of(sid * stripe, ZR)

    pltpu.sync_copy(zrow_hbm, zbuf)

    def zbody(k, c):
        off = pl.multiple_of(lo + k * ZR, ZR)
        pltpu.sync_copy(zbuf, agg_sh.at[pl.ds(off, ZR)])
        return c

    lax.fori_loop(0, stripe // ZR, zbody, 0)
    plsc.subcore_barrier()

    def chunk_body(c, carry):
        coff = pl.multiple_of(c * KI, KI)
        pltpu.sync_copy(src_hbm.at[wid, pl.ds(coff, KI)], idx_src)
        pltpu.sync_copy(dst_hbm.at[wid, pl.ds(coff, KI)], idx_dst)
        bufs = (rows0, rows1)
        sems = (sem0, sem1)
        descs = [None, None]
        descs[0] = pltpu.async_copy(x_hbm.at[idx_src.at[0]], rows0, sem0)
        for j in range(KI):
            if j + 1 < KI:
                descs[(j + 1) % 2] = pltpu.async_copy(
                    x_hbm.at[idx_src.at[j + 1]], bufs[(j + 1) % 2],
                    sems[(j + 1) % 2])
            descs[j % 2].wait()
            pltpu.sync_copy(bufs[j % 2], agg_sh.at[idx_dst.at[j]], add=True)
        return carry

    lax.fori_loop(0, nb // KI, chunk_body, 0)
    plsc.subcore_barrier()

    pltpu.sync_copy(agg_sh.at[pl.ds(lo, stripe)],
                    agg_out.at[cid, pl.ds(lo, stripe)])


def _make_sc_agg(nb, npad, d):
    mesh = plsc.VectorSubcoreMesh(core_axis_name="c", subcore_axis_name="s")
    return pl.kernel(
        functools.partial(_sc_agg_body, nb, npad, d),
        out_type=jax.ShapeDtypeStruct((NC, npad, d), jnp.float32),
        mesh=mesh,
        scratch_types=[
            pltpu.VMEM_SHARED((npad, d), jnp.float32),
            pltpu.VMEM((KI, B), jnp.int32),
            pltpu.VMEM((KI, B), jnp.int32),
            pltpu.VMEM((B, d), jnp.float32),
            pltpu.VMEM((B, d), jnp.float32),
            pltpu.VMEM((ZR, d), jnp.float32),
            pltpu.SemaphoreType.DMA,
            pltpu.SemaphoreType.DMA,
        ],
    )


def _sc_cnt_body(nb, npad, d, dst_hbm, zrow_hbm, ones_hbm,
                 cnt_out, cnt_sh, idx_dst, ones_v, zcbuf):
    cid = lax.axis_index("c")
    sid = lax.axis_index("s")
    wid = cid * NS + sid
    stripe = npad // NS
    lo = pl.multiple_of(sid * stripe, ZR)

    pltpu.sync_copy(zrow_hbm, zcbuf)
    pltpu.sync_copy(ones_hbm, ones_v)
    pltpu.sync_copy(dst_hbm.at[wid], idx_dst)

    def zbody(k, c):
        off = pl.multiple_of(lo + k * ZR, ZR)
        pltpu.sync_copy(zcbuf, cnt_sh.at[pl.ds(off, ZR)])
        return c

    lax.fori_loop(0, stripe // ZR, zbody, 0)
    plsc.subcore_barrier()

    def body(j, c2):
        pltpu.sync_copy(ones_v, cnt_sh.at[idx_dst.at[j]], add=True)
        return c2

    lax.fori_loop(0, nb, body, 0)
    plsc.subcore_barrier()
    pltpu.sync_copy(cnt_sh.at[pl.ds(lo, stripe)],
                    cnt_out.at[cid, pl.ds(lo, stripe)])


def _make_sc_cnt(nb, npad, d):
    mesh = plsc.VectorSubcoreMesh(core_axis_name="c", subcore_axis_name="s")
    return pl.kernel(
        functools.partial(_sc_cnt_body, nb, npad, d),
        out_type=jax.ShapeDtypeStruct((NC, npad, d), jnp.float32),
        mesh=mesh,
        scratch_types=[
            pltpu.VMEM_SHARED((npad, d), jnp.float32),
            pltpu.VMEM((nb, B), jnp.int32),
            pltpu.VMEM((B, d), jnp.float32),
            pltpu.VMEM((ZR, d), jnp.float32),
        ],
    )


def _tc_layer_body(relu, x_ref, aggp_ref, cntp_ref, ws_ref, wn_ref, b_ref,
                   out_ref):
    agg = aggp_ref[0] + aggp_ref[1]
    cnt = cntp_ref[0][:, 0:1] + cntp_ref[1][:, 0:1]
    mean = agg / jnp.maximum(cnt, 1.0)
    h = (jnp.dot(x_ref[...], ws_ref[...], preferred_element_type=jnp.float32)
         + jnp.dot(mean, wn_ref[...], preferred_element_type=jnp.float32)
         + b_ref[...])
    if relu:
        h = jnp.maximum(h, 0.0)
    out_ref[...] = h


def _tc_layer(x, agg_p, cnt_p, ws, wn, b, relu):
    n, d = x.shape
    blk = 2000 if n % 2000 == 0 else n
    grid = (n // blk,)
    return pl.pallas_call(
        functools.partial(_tc_layer_body, relu),
        grid=grid,
        in_specs=[
            pl.BlockSpec((blk, d), lambda i: (i, 0)),
            pl.BlockSpec((NC, blk, d), lambda i: (0, i, 0)),
            pl.BlockSpec((NC, blk, d), lambda i: (0, i, 0)),
            pl.BlockSpec((d, d), lambda i: (0, 0)),
            pl.BlockSpec((d, d), lambda i: (0, 0)),
            pl.BlockSpec((1, d), lambda i: (0, 0)),
        ],
        out_specs=pl.BlockSpec((blk, d), lambda i: (i, 0)),
        out_shape=jax.ShapeDtypeStruct((n, d), jnp.float32),
    )(x, agg_p, cnt_p, ws, wn, b.reshape(1, d))


def kernel(x, edge_index, Ws1, Wn1, b1, Ws2, Wn2, b2):
    n, d = x.shape
    e = edge_index.shape[1]
    per_tile = math.ceil(e / NW)
    nb = math.ceil(per_tile / B)
    nb = ((nb + KI - 1) // KI) * KI
    cap = NW * nb * B
    npad = ((n + 8 + NS * 8 - 1) // (NS * 8)) * (NS * 8)
    stripe = npad // NS

    src = edge_index[0]
    dst = edge_index[1]
    pad = cap - e
    src_r = jnp.pad(src, (0, pad)).reshape(NW, nb, B)
    dst_r = jnp.pad(dst, (0, pad), constant_values=npad - 1).reshape(NW, nb, B)

    zrow = jnp.zeros((ZR, d), jnp.float32)
    ones = jnp.ones((B, d), jnp.float32)

    sc_agg = _make_sc_agg(nb, npad, d)
    sc_cnt = _make_sc_cnt(nb, npad, d)

    cnt_p = sc_cnt(dst_r, zrow, ones)
    agg1_p = sc_agg(x, src_r, dst_r, zrow)
    h = _tc_layer(x, agg1_p, cnt_p, Ws1, Wn1, b1, relu=True)
    agg2_p = sc_agg(h, src_r, dst_r, zrow)
    out = _tc_layer(h, agg2_p, cnt_p, Ws2, Wn2, b2, relu=False)
    return out


# speedup vs baseline: 3.8152x; 1.0220x over previous
import functools
import math

import jax
import jax.numpy as jnp
from jax import lax
from jax.experimental import pallas as pl
from jax.experimental.pallas import tpu as pltpu
from jax.experimental.pallas import tpu_sc as plsc

B = 128
KI = 16
NC = 2
NS = 16
NW = NC * NS
ZR = 8


def _sc_agg_body(nb, npad, d, x_hbm, src_hbm, dst_hbm, zrow_hbm,
                 agg_out, agg_sh, idx_src, idx_dst, rows0, rows1, zbuf,
                 sem0, sem1, sem2, sem3):
    cid = lax.axis_index("c")
    sid = lax.axis_index("s")
    wid = cid * NS + sid
    stripe = npad // NS
    lo = pl.multiple_of(sid * stripe, ZR)

    pltpu.sync_copy(zrow_hbm, zbuf)

    def zbody(k, c):
        off = pl.multiple_of(lo + k * ZR, ZR)
        pltpu.sync_copy(zbuf, agg_sh.at[pl.ds(off, ZR)])
        return c

    lax.fori_loop(0, stripe // ZR, zbody, 0)
    plsc.subcore_barrier()

    def chunk_body(c, carry):
        coff = pl.multiple_of(c * KI, KI)
        pltpu.sync_copy(src_hbm.at[wid, pl.ds(coff, KI)], idx_src)
        pltpu.sync_copy(dst_hbm.at[wid, pl.ds(coff, KI)], idx_dst)
        bufs = (rows0, rows1)
        gsems = (sem0, sem1)
        ssems = (sem2, sem3)
        gd = [None, None]
        sd = [None, None]
        gd[0] = pltpu.async_copy(x_hbm.at[idx_src.at[0]], rows0, sem0)
        for j in range(KI):
            if j >= 1:
                sd[(j - 1) % 2].wait()
            if j + 1 < KI:
                gd[(j + 1) % 2] = pltpu.async_copy(
                    x_hbm.at[idx_src.at[j + 1]], bufs[(j + 1) % 2],
                    gsems[(j + 1) % 2])
            gd[j % 2].wait()
            sd[j % 2] = pltpu.async_copy(bufs[j % 2],
                                         agg_sh.at[idx_dst.at[j]],
                                         ssems[j % 2], add=True)
        sd[(KI - 1) % 2].wait()
        return carry

    lax.fori_loop(0, nb // KI, chunk_body, 0)
    plsc.subcore_barrier()

    pltpu.sync_copy(agg_sh.at[pl.ds(lo, stripe)],
                    agg_out.at[cid, pl.ds(lo, stripe)])


def _make_sc_agg(nb, npad, d):
    mesh = plsc.VectorSubcoreMesh(core_axis_name="c", subcore_axis_name="s")
    return pl.kernel(
        functools.partial(_sc_agg_body, nb, npad, d),
        out_type=jax.ShapeDtypeStruct((NC, npad, d), jnp.float32),
        mesh=mesh,
        scratch_types=[
            pltpu.VMEM_SHARED((npad, d), jnp.float32),
            pltpu.VMEM((KI, B), jnp.int32),
            pltpu.VMEM((KI, B), jnp.int32),
            pltpu.VMEM((B, d), jnp.float32),
            pltpu.VMEM((B, d), jnp.float32),
            pltpu.VMEM((ZR, d), jnp.float32),
            pltpu.SemaphoreType.DMA,
            pltpu.SemaphoreType.DMA,
            pltpu.SemaphoreType.DMA,
            pltpu.SemaphoreType.DMA,
        ],
    )


def _sc_cnt_body(nb, npad, d, dst_hbm, zrow_hbm, ones_hbm,
                 cnt_out, cnt_sh, idx_dst, ones_v, zcbuf):
    cid = lax.axis_index("c")
    sid = lax.axis_index("s")
    wid = cid * NS + sid
    stripe = npad // NS
    lo = pl.multiple_of(sid * stripe, ZR)

    pltpu.sync_copy(zrow_hbm, zcbuf)
    pltpu.sync_copy(ones_hbm, ones_v)
    pltpu.sync_copy(dst_hbm.at[wid], idx_dst)

    def zbody(k, c):
        off = pl.multiple_of(lo + k * ZR, ZR)
        pltpu.sync_copy(zcbuf, cnt_sh.at[pl.ds(off, ZR)])
        return c

    lax.fori_loop(0, stripe // ZR, zbody, 0)
    plsc.subcore_barrier()

    def body(j, c2):
        pltpu.sync_copy(ones_v, cnt_sh.at[idx_dst.at[j]], add=True)
        return c2

    lax.fori_loop(0, nb, body, 0)
    plsc.subcore_barrier()
    pltpu.sync_copy(cnt_sh.at[pl.ds(lo, stripe)],
                    cnt_out.at[cid, pl.ds(lo, stripe)])


def _make_sc_cnt(nb, npad, d):
    mesh = plsc.VectorSubcoreMesh(core_axis_name="c", subcore_axis_name="s")
    return pl.kernel(
        functools.partial(_sc_cnt_body, nb, npad, d),
        out_type=jax.ShapeDtypeStruct((NC, npad, d), jnp.float32),
        mesh=mesh,
        scratch_types=[
            pltpu.VMEM_SHARED((npad, d), jnp.float32),
            pltpu.VMEM((nb, B), jnp.int32),
            pltpu.VMEM((B, d), jnp.float32),
            pltpu.VMEM((ZR, d), jnp.float32),
        ],
    )


def _tc_layer_body(relu, x_ref, aggp_ref, cntp_ref, ws_ref, wn_ref, b_ref,
                   out_ref):
    agg = aggp_ref[0] + aggp_ref[1]
    cnt = cntp_ref[0][:, 0:1] + cntp_ref[1][:, 0:1]
    mean = agg / jnp.maximum(cnt, 1.0)
    h = (jnp.dot(x_ref[...], ws_ref[...], preferred_element_type=jnp.float32)
         + jnp.dot(mean, wn_ref[...], preferred_element_type=jnp.float32)
         + b_ref[...])
    if relu:
        h = jnp.maximum(h, 0.0)
    out_ref[...] = h


def _tc_layer(x, agg_p, cnt_p, ws, wn, b, relu):
    n, d = x.shape
    blk = 2000 if n % 2000 == 0 else n
    grid = (n // blk,)
    return pl.pallas_call(
        functools.partial(_tc_layer_body, relu),
        grid=grid,
        in_specs=[
            pl.BlockSpec((blk, d), lambda i: (i, 0)),
            pl.BlockSpec((NC, blk, d), lambda i: (0, i, 0)),
            pl.BlockSpec((NC, blk, d), lambda i: (0, i, 0)),
            pl.BlockSpec((d, d), lambda i: (0, 0)),
            pl.BlockSpec((d, d), lambda i: (0, 0)),
            pl.BlockSpec((1, d), lambda i: (0, 0)),
        ],
        out_specs=pl.BlockSpec((blk, d), lambda i: (i, 0)),
        out_shape=jax.ShapeDtypeStruct((n, d), jnp.float32),
    )(x, agg_p, cnt_p, ws, wn, b.reshape(1, d))


def kernel(x, edge_index, Ws1, Wn1, b1, Ws2, Wn2, b2):
    n, d = x.shape
    e = edge_index.shape[1]
    per_tile = math.ceil(e / NW)
    nb = math.ceil(per_tile / B)
    nb = ((nb + KI - 1) // KI) * KI
    cap = NW * nb * B
    npad = ((n + 8 + NS * 8 - 1) // (NS * 8)) * (NS * 8)
    stripe = npad // NS

    src = edge_index[0]
    dst = edge_index[1]
    pad = cap - e
    src_r = jnp.pad(src, (0, pad)).reshape(NW, nb, B)
    dst_r = jnp.pad(dst, (0, pad), constant_values=npad - 1).reshape(NW, nb, B)

    zrow = jnp.zeros((ZR, d), jnp.float32)
    ones = jnp.ones((B, d), jnp.float32)

    sc_agg = _make_sc_agg(nb, npad, d)
    sc_cnt = _make_sc_cnt(nb, npad, d)

    cnt_p = sc_cnt(dst_r, zrow, ones)
    agg1_p = sc_agg(x, src_r, dst_r, zrow)
    h = _tc_layer(x, agg1_p, cnt_p, Ws1, Wn1, b1, relu=True)
    agg2_p = sc_agg(h, src_r, dst_r, zrow)
    out = _tc_layer(h, agg2_p, cnt_p, Ws2, Wn2, b2, relu=False)
    return out
